# Initial kernel scaffold; baseline (speedup 1.0000x reference)
import jax
import jax.numpy as jnp
from jax.experimental import pallas as pl


def kernel(w_tensor, table):
    raise NotImplementedError("write your pallas kernel here")


# speedup vs baseline: 1.4576x; 1.4576x over previous
import functools

import jax
import jax.numpy as jnp
from jax import lax
from jax.experimental import pallas as pl
from jax.experimental.pallas import tpu as pltpu
from jax.experimental.pallas import tpu_sc as plsc

VOCAB = 1000000
EMBED_DIM = 32
BATCH = 4096
HIST = 200
TOTAL = BATCH * HIST

NUM_CORES = 2
NUM_SUBCORES = 16
NW = NUM_CORES * NUM_SUBCORES
PER_W = TOTAL // NW

IDX_ROW = 128
K = 8
CHUNK = K * IDX_ROW
NCHUNK = PER_W // CHUNK
assert NCHUNK * CHUNK == PER_W

_mesh = plsc.VectorSubcoreMesh(
    core_axis_name="c", subcore_axis_name="s",
    num_cores=NUM_CORES, num_subcores=NUM_SUBCORES)


@functools.partial(
    pl.kernel,
    out_type=jax.ShapeDtypeStruct((TOTAL, EMBED_DIM), jnp.float32),
    mesh=_mesh,
    scratch_types=[
        pltpu.VMEM((K, IDX_ROW), jnp.int32),
        pltpu.VMEM((CHUNK, EMBED_DIM), jnp.float32),
        pltpu.SemaphoreType.DMA,
    ],
    compiler_params=pltpu.CompilerParams(use_tc_tiling_on_sc=False),
)
def _emb_lookup(idx_hbm, table_hbm, out_hbm, idx_v, rows_v, gsem):
    wid = lax.axis_index("s") * NUM_CORES + lax.axis_index("c")
    row_base = wid * (PER_W // IDX_ROW)

    @pl.loop(0, NCHUNK)
    def _chunk(g):
        crow = pl.multiple_of(row_base + g * K, 8)
        pltpu.sync_copy(idx_hbm.at[pl.ds(crow, K)], idx_v)
        copies = [
            pltpu.async_copy(
                table_hbm.at[idx_v.at[j]],
                rows_v.at[pl.ds(j * IDX_ROW, IDX_ROW)],
                gsem,
            )
            for j in range(K)
        ]
        for c in copies:
            c.wait()
        pltpu.sync_copy(rows_v, out_hbm.at[pl.ds(crow * IDX_ROW, CHUNK)])


@jax.jit
def kernel(w_tensor, table):
    idx = w_tensor.reshape(TOTAL // IDX_ROW, IDX_ROW).astype(jnp.int32)
    out = _emb_lookup(idx, table)
    return out.reshape(BATCH, HIST, EMBED_DIM)


# speedup vs baseline: 1.4860x; 1.0195x over previous
import functools

import jax
import jax.numpy as jnp
from jax import lax
from jax.experimental import pallas as pl
from jax.experimental.pallas import tpu as pltpu
from jax.experimental.pallas import tpu_sc as plsc

VOCAB = 1000000
EMBED_DIM = 32
BATCH = 4096
HIST = 200
TOTAL = BATCH * HIST

NUM_CORES = 2
NUM_SUBCORES = 16
NW = NUM_CORES * NUM_SUBCORES
PER_W = TOTAL // NW

IDX_ROW = 128
K = 8
CHUNK = K * IDX_ROW
NCHUNK = PER_W // CHUNK
assert NCHUNK * CHUNK == PER_W

_mesh = plsc.VectorSubcoreMesh(
    core_axis_name="c", subcore_axis_name="s",
    num_cores=NUM_CORES, num_subcores=NUM_SUBCORES)


@functools.partial(
    pl.kernel,
    out_type=jax.ShapeDtypeStruct((TOTAL, EMBED_DIM), jnp.float32),
    mesh=_mesh,
    scratch_types=[
        pltpu.VMEM((2, K, IDX_ROW), jnp.int32),
        pltpu.VMEM((2, CHUNK, EMBED_DIM), jnp.float32),
        pltpu.SemaphoreType.DMA,
        pltpu.SemaphoreType.DMA,
    ],
    compiler_params=pltpu.CompilerParams(use_tc_tiling_on_sc=False),
)
def _emb_lookup(idx_hbm, table_hbm, out_hbm, idx_v, rows_v, gsem, osem):
    wid = lax.axis_index("s") * NUM_CORES + lax.axis_index("c")
    row_base = wid * (PER_W // IDX_ROW)

    def stage(c, buf):
        crow = pl.multiple_of(row_base + c * K, 8)
        pltpu.sync_copy(idx_hbm.at[pl.ds(crow, K)], idx_v.at[buf])
        for j in range(K):
            pltpu.async_copy(
                table_hbm.at[idx_v.at[buf, j]],
                rows_v.at[buf, pl.ds(j * IDX_ROW, IDX_ROW)],
                gsem,
            )

    def drain_gathers(buf):
        pltpu.make_async_copy(
            table_hbm.at[pl.ds(0, CHUNK)], rows_v.at[buf], gsem).wait()

    def writeback(c, buf):
        crow = pl.multiple_of(row_base + c * K, 8)
        pltpu.async_copy(
            rows_v.at[buf], out_hbm.at[pl.ds(crow * IDX_ROW, CHUNK)], osem)

    def drain_out(buf):
        pltpu.make_async_copy(
            rows_v.at[buf], out_hbm.at[pl.ds(0, CHUNK)], osem).wait()

    stage(0, 0)
    stage(1, 1)
    drain_gathers(0)
    writeback(0, 0)

    @pl.loop(0, (NCHUNK - 1) // 2)
    def _pair(p):
        c0 = 1 + 2 * p
        for t in range(2):
            c = c0 + t
            buf = (1 + t) % 2
            nbuf = t % 2

            @pl.when(c + 1 < NCHUNK)
            def _prefetch():
                drain_out(nbuf)
                stage(c + 1, nbuf)

            drain_gathers(buf)
            writeback(c, buf)

    drain_out(0)
    drain_out(1)


@jax.jit
def kernel(w_tensor, table):
    idx = w_tensor.reshape(TOTAL // IDX_ROW, IDX_ROW).astype(jnp.int32)
    out = _emb_lookup(idx, table)
    return out.reshape(BATCH, HIST, EMBED_DIM)
